# baseline (speedup 1.0000x reference)
import functools

import jax
import jax.numpy as jnp
from jax import lax
from jax.experimental import pallas as pl
from jax.experimental.pallas import tpu as pltpu
from jax.experimental.pallas import tpu_sc as plsc

F = 26
B = 4096
L = 20
V = 100000
D = 16

NC = 2
NS = 16
NW = NC * NS

NB = 16
NF = 2
FPW = F // NF
R = B // NB
IDX_PER_CHUNK = R * L
GROWS = 128
NG = IDX_PER_CHUNK // GROWS
WIN = 5
NWIN = NG // WIN
BAGS_PER_WIN = WIN * GROWS // L
IDX_ROWS = B * L // GROWS

_mesh = plsc.VectorSubcoreMesh(
    core_axis_name="c", subcore_axis_name="s", num_cores=NC, num_subcores=NS
)


@functools.partial(
    pl.kernel,
    out_type=jax.ShapeDtypeStruct((B, F, D), jnp.float32),
    mesh=_mesh,
    scratch_types=[
        pltpu.VMEM((NG, GROWS), jnp.int32),
        pltpu.VMEM((IDX_PER_CHUNK, D), jnp.float32),
        pltpu.VMEM((R, D), jnp.float32),
        pltpu.SemaphoreType.DMA((NWIN,)),
    ],
    compiler_params=pltpu.CompilerParams(use_tc_tiling_on_sc=False),
)
def _pooled_gather(tab_hbm, idx_hbm, out_hbm, idx_v, g_v, o_v, sems):
    wid = lax.axis_index("s") * NC + lax.axis_index("c")
    fh = wid // NB
    bc = wid - fh * NB
    row0 = pl.multiple_of(bc * NG, 8)
    bag0 = pl.multiple_of(bc * R, 8)

    @pl.loop(0, FPW)
    def _feat(fi):
        f = fh * FPW + fi
        pltpu.sync_copy(idx_hbm.at[pl.ds(f * IDX_ROWS + row0, NG)], idx_v)
        foff = f * V

        @pl.loop(0, NG)
        def _off(r):
            for k in range(GROWS // 16):
                sl = pl.ds(k * 16, 16)
                idx_v[r, sl] = idx_v[r, sl] + foff

        copies = [
            pltpu.async_copy(
                tab_hbm.at[idx_v.at[j]],
                g_v.at[pl.ds(j * GROWS, GROWS)],
                sems.at[j // WIN],
            )
            for j in range(NG)
        ]
        for w in range(NWIN):
            for cp in copies[w * WIN : (w + 1) * WIN]:
                cp.wait()

            @pl.loop(0, BAGS_PER_WIN)
            def _bag(b):
                base = (w * BAGS_PER_WIN + b) * L
                acc = g_v[base, :]
                for l in range(1, L):
                    acc = acc + g_v[base + l, :]
                o_v[w * BAGS_PER_WIN + b, :] = acc

        pltpu.sync_copy(o_v, out_hbm.at[pl.ds(bag0, R), f])


def kernel(indices, tables):
    idx2 = indices.reshape(F * IDX_ROWS, GROWS)
    pooled = _pooled_gather(tables.reshape(F * V, D), idx2)
    return pooled.reshape(B, F * D)


# speedup vs baseline: 1.0387x; 1.0387x over previous
import functools

import jax
import jax.numpy as jnp
from jax import lax
from jax.experimental import pallas as pl
from jax.experimental.pallas import tpu as pltpu
from jax.experimental.pallas import tpu_sc as plsc

F = 26
B = 4096
L = 20
V = 100000
D = 16

NC = 2
NS = 16
NW = NC * NS

NB = 16
NF = 2
FPW = F // NF
R = B // NB
IDX_PER_CHUNK = R * L
GROWS = 128
NG = IDX_PER_CHUNK // GROWS
WIN = 5
NWIN = NG // WIN
BAGS_PER_WIN = WIN * GROWS // L
IDX_ROWS = B * L // GROWS

_mesh = plsc.VectorSubcoreMesh(
    core_axis_name="c", subcore_axis_name="s", num_cores=NC, num_subcores=NS
)


@functools.partial(
    pl.kernel,
    out_type=jax.ShapeDtypeStruct((B, F * D), jnp.float32),
    mesh=_mesh,
    scratch_types=[
        pltpu.VMEM((NG, GROWS), jnp.int32),
        pltpu.VMEM((IDX_PER_CHUNK, D), jnp.float32),
        pltpu.VMEM((R, D), jnp.float32),
        pltpu.SemaphoreType.DMA((NWIN,)),
    ],
    compiler_params=pltpu.CompilerParams(use_tc_tiling_on_sc=False),
)
def _pooled_gather(tab3_hbm, idx3_hbm, out_hbm, idx_v, g_v, o_v, sems):
    wid = lax.axis_index("s") * NC + lax.axis_index("c")
    fh = wid // NB
    bc = wid - fh * NB
    row0 = pl.multiple_of(bc * NG, 8)
    bag0 = pl.multiple_of(bc * R, 8)

    @pl.loop(0, FPW)
    def _feat(fi):
        f = fh * FPW + fi
        pltpu.sync_copy(idx3_hbm.at[f, pl.ds(row0, NG), :], idx_v)
        tab_f = tab3_hbm.at[f]
        copies = [
            pltpu.async_copy(
                tab_f.at[idx_v.at[j]],
                g_v.at[pl.ds(j * GROWS, GROWS)],
                sems.at[j // WIN],
            )
            for j in range(NG)
        ]
        for w in range(NWIN):
            for cp in copies[w * WIN : (w + 1) * WIN]:
                cp.wait()

            @pl.loop(0, BAGS_PER_WIN)
            def _bag(b):
                base = (w * BAGS_PER_WIN + b) * L
                acc = g_v[base, :]
                for l in range(1, L):
                    acc = acc + g_v[base + l, :]
                o_v[w * BAGS_PER_WIN + b, :] = acc

        pltpu.sync_copy(o_v, out_hbm.at[pl.ds(bag0, R), pl.ds(f * D, D)])


def kernel(indices, tables):
    return _pooled_gather(tables, indices.reshape(F, IDX_ROWS, GROWS))
